# baseline (speedup 1.0000x reference)
import functools

import jax
import jax.numpy as jnp
from jax import lax
from jax.experimental import pallas as pl
from jax.experimental.pallas import tpu as pltpu
from jax.experimental.pallas import tpu_sc as plsc

BATCH = 16384
LATENT = 64
HIDDEN = 32
PAIR = 2 * LATENT

NUM_CORES = 2
NUM_SUBCORES = 16
NUM_WORKERS = NUM_CORES * NUM_SUBCORES
ROWS_PER_W = BATCH // NUM_WORKERS
IDX_CHUNK = 128
NUM_CHUNKS = ROWS_PER_W // IDX_CHUNK

DENSE_BLOCK = 2048


@functools.cache
def _build_gather(n_pair_rows):
    mesh = plsc.VectorSubcoreMesh(core_axis_name="c", subcore_axis_name="s")

    @functools.partial(
        pl.kernel,
        mesh=mesh,
        out_type=jax.ShapeDtypeStruct((BATCH, PAIR), jnp.float32),
        scratch_types=[
            pltpu.VMEM((ROWS_PER_W,), jnp.int32),
            pltpu.VMEM((ROWS_PER_W, PAIR), jnp.float32),
            pltpu.SemaphoreType.DMA,
        ],
    )
    def gather(table_hbm, idx_hbm, out_hbm, idx_v, rows_v, sem):
        wid = lax.axis_index("s") * NUM_CORES + lax.axis_index("c")
        base = wid * ROWS_PER_W
        pltpu.sync_copy(idx_hbm.at[pl.ds(base, ROWS_PER_W)], idx_v)
        copies = [
            pltpu.make_async_copy(
                table_hbm.at[idx_v.at[pl.ds(j * IDX_CHUNK, IDX_CHUNK)]],
                rows_v.at[pl.ds(j * IDX_CHUNK, IDX_CHUNK)],
                sem,
            )
            for j in range(NUM_CHUNKS)
        ]
        for c in copies:
            c.start()
        for c in copies:
            c.wait()
        pltpu.sync_copy(rows_v, out_hbm.at[pl.ds(base, ROWS_PER_W)])

    return gather


def _dense_body(pair_ref, par_ref, wt_ref, b_ref, out_ref):
    pair = pair_ref[...]
    par = par_ref[...]
    emb = pair[:, :LATENT] * (1.0 - par) + pair[:, LATENT:] * par
    h = jnp.dot(emb, wt_ref[...], preferred_element_type=jnp.float32)
    h = h + b_ref[...]
    ones = jnp.ones((HIDDEN, 1), jnp.float32)
    out_ref[...] = jnp.dot(h * h, ones, preferred_element_type=jnp.float32)


@functools.cache
def _build_dense():
    return pl.pallas_call(
        _dense_body,
        grid=(BATCH // DENSE_BLOCK,),
        in_specs=[
            pl.BlockSpec((DENSE_BLOCK, PAIR), lambda i: (i, 0)),
            pl.BlockSpec((DENSE_BLOCK, 1), lambda i: (i, 0)),
            pl.BlockSpec((LATENT, HIDDEN), lambda i: (0, 0)),
            pl.BlockSpec((1, HIDDEN), lambda i: (0, 0)),
        ],
        out_specs=pl.BlockSpec((DENSE_BLOCK, 1), lambda i: (i, 0)),
        out_shape=jax.ShapeDtypeStruct((BATCH, 1), jnp.float32),
    )


def kernel(user_ids, item_ids, user_table, item_table, W_user, b_user, W_item, b_item):
    del item_ids, item_table, W_item, b_item
    ids = user_ids.astype(jnp.int32)
    pair_ids = lax.shift_right_logical(ids, 1)
    parity = (ids & 1).astype(jnp.float32).reshape(BATCH, 1)
    n_pair_rows = user_table.shape[0] // 2
    table_pairs = user_table.reshape(n_pair_rows, PAIR)
    pairs = _build_gather(n_pair_rows)(table_pairs, pair_ids)
    out = _build_dense()(pairs, parity, W_user.T, b_user.reshape(1, HIDDEN))
    return out.reshape(BATCH)


# speedup vs baseline: 6.3917x; 6.3917x over previous
import functools

import jax
import jax.numpy as jnp
from jax import lax
from jax.experimental import pallas as pl
from jax.experimental.pallas import tpu as pltpu
from jax.experimental.pallas import tpu_sc as plsc

BATCH = 16384
LATENT = 64
HIDDEN = 32
LANES = 128

N_ROWS = 1000000
SCAN_BLK = 32768
N_BLOCKS = -(-N_ROWS // SCAN_BLK)
Q_PAD = N_BLOCKS * SCAN_BLK
Q_ROWS = Q_PAD // LANES

NUM_CORES = 2
NUM_SUBCORES = 16
NUM_WORKERS = NUM_CORES * NUM_SUBCORES
IDS_PER_W = BATCH // NUM_WORKERS
IDX_CHUNK = 128
NUM_CHUNKS = IDS_PER_W // IDX_CHUNK
VL = 16


def _scan_body(tT_ref, w_ref, b_ref, q_ref):
    h = jnp.dot(w_ref[...], tT_ref[...], preferred_element_type=jnp.float32)
    h = h + b_ref[...]
    q = jnp.sum(h * h, axis=0)
    q_ref[...] = jnp.reshape(q, (SCAN_BLK // LANES, LANES))


@functools.cache
def _build_scan():
    return pl.pallas_call(
        _scan_body,
        grid=(N_BLOCKS,),
        in_specs=[
            pl.BlockSpec((LATENT, SCAN_BLK), lambda i: (0, i)),
            pl.BlockSpec((HIDDEN, LATENT), lambda i: (0, 0)),
            pl.BlockSpec((HIDDEN, 1), lambda i: (0, 0)),
        ],
        out_specs=pl.BlockSpec((SCAN_BLK // LANES, LANES), lambda i: (i, 0)),
        out_shape=jax.ShapeDtypeStruct((Q_ROWS, LANES), jnp.float32),
    )


@functools.cache
def _build_extract():
    mesh = plsc.VectorSubcoreMesh(core_axis_name="c", subcore_axis_name="s")

    @functools.partial(
        pl.kernel,
        mesh=mesh,
        compiler_params=pltpu.CompilerParams(needs_layout_passes=False),
        out_type=jax.ShapeDtypeStruct((BATCH,), jnp.float32),
        scratch_types=[
            pltpu.VMEM((IDS_PER_W,), jnp.int32),
            pltpu.VMEM((IDS_PER_W,), jnp.int32),
            pltpu.VMEM((IDS_PER_W, LANES), jnp.float32),
            pltpu.VMEM((IDS_PER_W,), jnp.float32),
            pltpu.SemaphoreType.DMA,
        ],
    )
    def extract(q_hbm, idx_hbm, out_hbm, idx_v, hi_v, rows_v, res_v, sem):
        wid = lax.axis_index("s") * NUM_CORES + lax.axis_index("c")
        base = wid * IDS_PER_W
        pltpu.sync_copy(idx_hbm.at[pl.ds(base, IDS_PER_W)], idx_v)
        for k in range(IDS_PER_W // VL):
            hi_v[pl.ds(k * VL, VL)] = lax.shift_right_logical(
                idx_v[pl.ds(k * VL, VL)], 7
            )
        copies = [
            pltpu.make_async_copy(
                q_hbm.at[hi_v.at[pl.ds(j * IDX_CHUNK, IDX_CHUNK)]],
                rows_v.at[pl.ds(j * IDX_CHUNK, IDX_CHUNK)],
                sem,
            )
            for j in range(NUM_CHUNKS)
        ]
        for c in copies:
            c.start()
        for c in copies:
            c.wait()
        for k in range(IDS_PER_W // VL):
            row_ids = lax.iota(jnp.int32, VL) + (k * VL)
            lane_ids = idx_v[pl.ds(k * VL, VL)] & (LANES - 1)
            res_v[pl.ds(k * VL, VL)] = plsc.load_gather(
                rows_v, [row_ids, lane_ids]
            )
        pltpu.sync_copy(res_v, out_hbm.at[pl.ds(base, IDS_PER_W)])

    return extract


def kernel(user_ids, item_ids, user_table, item_table, W_user, b_user, W_item, b_item):
    del item_ids, item_table, W_item, b_item
    ids = user_ids.astype(jnp.int32)
    tableT = user_table.T
    q = _build_scan()(tableT, W_user, b_user.reshape(HIDDEN, 1))
    return _build_extract()(q, ids)
